# baseline (speedup 1.0000x reference)
import functools

import jax
import jax.numpy as jnp
from jax import lax
from jax.experimental import pallas as pl
from jax.experimental.pallas import tpu as pltpu
from jax.experimental.pallas import tpu_sc as plsc

_N = 10000
_E = 320000
_H = 128
_O = 40
_EPS = 0.3
_NPAD = 10112
_NW = 32
_K = 64
_EPT = 10112
_EPAD = _NW * _EPT
_NFC = _EPT // _K
_RPW = _NPAD // 16
_BLK = 632
_NBLK = _NPAD // _BLK


@functools.cache
def _sc_kernels():
    mesh = plsc.VectorSubcoreMesh(core_axis_name="c", subcore_axis_name="s",
                                  num_cores=2, num_subcores=16)
    params = pltpu.CompilerParams(needs_layout_passes=False)
    deg = functools.partial(
        pl.kernel,
        out_type=jax.ShapeDtypeStruct((2, _NPAD, _H), jnp.float32),
        mesh=mesh,
        compiler_params=params,
        scratch_types=[
            pltpu.VMEM((_NFC, _K), jnp.int32),
            pltpu.VMEM((_K, _H), jnp.float32),
            pltpu.VMEM_SHARED((_NPAD, _H), jnp.float32),
        ],
    )(_deg_body)
    layer = functools.partial(
        pl.kernel,
        out_type=jax.ShapeDtypeStruct((2, _NPAD, _H), jnp.float32),
        mesh=mesh,
        compiler_params=params,
        scratch_types=[
            pltpu.VMEM((_NPAD,), jnp.float32),
            pltpu.VMEM((_NPAD,), jnp.float32),
            pltpu.VMEM((2, _K), jnp.int32),
            pltpu.VMEM((2, _K), jnp.int32),
            pltpu.VMEM((_K,), jnp.float32),
            pltpu.VMEM((_K, _H), jnp.float32),
            pltpu.VMEM((_K, _H), jnp.float32),
            pltpu.SemaphoreType.DMA,
            pltpu.SemaphoreType.DMA,
            pltpu.SemaphoreType.DMA,
            pltpu.SemaphoreType.DMA,
            pltpu.VMEM_SHARED((_NPAD, _H), jnp.float32),
        ],
    )(_layer_body)
    return deg, layer


def _deg_body(dst_hbm, out_hbm, idx_v, ones_v, deg_sh):
    c = lax.axis_index("c")
    s = lax.axis_index("s")
    wid = s * 2 + c
    pltpu.sync_copy(dst_hbm.at[wid], idx_v)

    def _fill(val):
        def _body(i, carry):
            for cb in range(_H // 16):
                ones_v[i, pl.ds(cb * 16, 16)] = jnp.full((16,), val,
                                                         jnp.float32)
            return carry
        lax.fori_loop(0, _K, _body, 0)

    row0 = s * _RPW
    _fill(0.0)

    def _zero(j, carry):
        pltpu.sync_copy(ones_v, deg_sh.at[pl.ds(row0 + j * _K, _K)])
        return carry

    lax.fori_loop(0, _RPW // _K, _zero, 0)
    pltpu.sync_copy(ones_v.at[pl.ds(0, _RPW % _K)],
                    deg_sh.at[pl.ds(row0 + (_RPW // _K) * _K, _RPW % _K)])
    _fill(1.0)
    plsc.subcore_barrier()

    def _scat(g, carry):
        pltpu.sync_copy(ones_v, deg_sh.at[idx_v.at[g]], add=True)
        return carry

    lax.fori_loop(0, _NFC, _scat, 0)
    plsc.subcore_barrier()
    pltpu.sync_copy(deg_sh.at[pl.ds(row0, _RPW)],
                    out_hbm.at[c, pl.ds(row0, _RPW)])


def _layer_body(src_hbm, dst_hbm, a_hbm, b_hbm, hd_hbm, out_hbm,
                a_v, b_v, idx0, idx1, e_v, rows0, rows1,
                gsem0, gsem1, ssem0, ssem1, z_sh):
    c = lax.axis_index("c")
    s = lax.axis_index("s")
    wid = s * 2 + c
    ebase = wid * _EPT
    pltpu.sync_copy(a_hbm, a_v)
    pltpu.sync_copy(b_hbm, b_v)

    def _zrow(i, carry):
        for cb in range(_H // 16):
            rows0[i, pl.ds(cb * 16, 16)] = jnp.zeros((16,), jnp.float32)
        return carry

    lax.fori_loop(0, _K, _zrow, 0)
    row0 = s * _RPW

    def _zcp(j, carry):
        pltpu.sync_copy(rows0, z_sh.at[pl.ds(row0 + j * _K, _K)])
        return carry

    lax.fori_loop(0, _RPW // _K, _zcp, 0)
    pltpu.sync_copy(rows0.at[pl.ds(0, _RPW % _K)],
                    z_sh.at[pl.ds(row0 + (_RPW // _K) * _K, _RPW % _K)])

    def _fetch(cc, idx, rows, sem):
        pltpu.sync_copy(src_hbm.at[pl.ds(ebase + cc * _K, _K)], idx.at[0])
        pltpu.sync_copy(dst_hbm.at[pl.ds(ebase + cc * _K, _K)], idx.at[1])
        pltpu.async_copy(hd_hbm.at[idx.at[0]], rows, sem)

    _fetch(0, idx0, rows0, gsem0)
    _fetch(1, idx1, rows1, gsem1)
    plsc.subcore_barrier()

    def _gate(idx):
        for j in range(_K // 16):
            sv = idx[0, pl.ds(j * 16, 16)]
            dv = idx[1, pl.ds(j * 16, 16)]
            av = plsc.load_gather(a_v, [dv])
            bv = plsc.load_gather(b_v, [sv])
            x = av + bv
            t = jnp.exp(-2.0 * jnp.abs(x))
            e_v[pl.ds(j * 16, 16)] = jnp.sign(x) * (1.0 - t) / (1.0 + t)

    def _scale(rows):
        def _body(q, carry):
            for u in range(8):
                k = q * 8 + u
                ek = plsc.load_gather(e_v, [jnp.full((16,), k, jnp.int32)])
                for cb in range(_H // 16):
                    rows[k, pl.ds(cb * 16, 16)] = (
                        rows[k, pl.ds(cb * 16, 16)] * ek)
            return carry

        lax.fori_loop(0, _K // 8, _body, 0)

    def _pair(t, carry):
        c0 = 2 * t
        _gate(idx0)
        pltpu.make_async_copy(hd_hbm.at[idx0.at[0]], rows0, gsem0).wait()
        _scale(rows0)
        pltpu.sync_copy(rows0, z_sh.at[idx0.at[1]], add=True)

        @pl.when(c0 + 2 < _NFC)
        def _():
            _fetch(c0 + 2, idx0, rows0, gsem0)

        _gate(idx1)
        pltpu.make_async_copy(hd_hbm.at[idx1.at[0]], rows1, gsem1).wait()
        _scale(rows1)
        pltpu.sync_copy(rows1, z_sh.at[idx1.at[1]], add=True)

        @pl.when(c0 + 3 < _NFC)
        def _():
            _fetch(c0 + 3, idx1, rows1, gsem1)

        return carry

    lax.fori_loop(0, _NFC // 2, _pair, 0)

    plsc.subcore_barrier()
    pltpu.sync_copy(z_sh.at[pl.ds(row0, _RPW)],
                    out_hbm.at[c, pl.ds(row0, _RPW)])


def _tc1_body(h_ref, w1_ref, b1_ref, wg_ref, bg_ref, deg_ref,
              h1_ref, hd_ref, a_ref, b_ref, d_ref):
    x = h_ref[...]
    h1 = jnp.maximum(
        jnp.dot(x, w1_ref[...], preferred_element_type=jnp.float32)
        + b1_ref[...], 0.0)
    degb = deg_ref[...]
    deg = degb[0, :, 0:1] + degb[1, :, 0:1]
    d = jnp.where(deg > 0.0, lax.rsqrt(jnp.maximum(deg, 1.0)), 0.0)
    h1_ref[...] = h1
    hd_ref[...] = h1 * d
    d_ref[...] = d
    ab = jnp.dot(h1, wg_ref[...],
                 preferred_element_type=jnp.float32) + bg_ref[...]
    a_ref[...] = ab[:, 0:1]
    b_ref[...] = ab[:, 1:2]


def _tc2_body(zp_ref, h1_ref, d_ref, wg_ref, bg_ref, hd_ref, a_ref, b_ref):
    z = zp_ref[0] + zp_ref[1]
    d = d_ref[...]
    h2 = _EPS * h1_ref[...] + d * z
    hd_ref[...] = h2 * d
    ab = jnp.dot(h2, wg_ref[...],
                 preferred_element_type=jnp.float32) + bg_ref[...]
    a_ref[...] = ab[:, 0:1]
    b_ref[...] = ab[:, 1:2]


def _tc3_body(zp_ref, h1_ref, d_ref, w2_ref, b2_ref, out_ref):
    z = zp_ref[0] + zp_ref[1]
    h3 = _EPS * h1_ref[...] + d_ref[...] * z
    logits = jnp.dot(h3, w2_ref[...],
                     preferred_element_type=jnp.float32) + b2_ref[...]
    col = lax.broadcasted_iota(jnp.int32, (_BLK, _H), 1)
    logits = jnp.where(col < _O, logits, -1e30)
    m = jnp.max(logits, axis=1, keepdims=True)
    lse = jnp.log(jnp.sum(jnp.exp(logits - m), axis=1, keepdims=True))
    out_ref[...] = logits - m - lse


_row_spec = pl.BlockSpec((_BLK, _H), lambda i: (i, 0))
_w_spec = pl.BlockSpec((_H, _H), lambda i: (0, 0))
_bias_spec = pl.BlockSpec((1, _H), lambda i: (0, 0))
_w16_spec = pl.BlockSpec((_H, 16), lambda i: (0, 0))
_b16_spec = pl.BlockSpec((1, 16), lambda i: (0, 0))
_d_spec = pl.BlockSpec((_BLK, 1), lambda i: (i, 0))
_zp_spec = pl.BlockSpec((2, _BLK, _H), lambda i: (0, i, 0))
_fmat = jax.ShapeDtypeStruct((_NPAD, _H), jnp.float32)
_fcol = jax.ShapeDtypeStruct((_NPAD, 1), jnp.float32)

_tc1_call = pl.pallas_call(
    _tc1_body,
    grid=(_NBLK,),
    in_specs=[_row_spec, _w_spec, _bias_spec, _w16_spec, _b16_spec,
              _zp_spec],
    out_specs=[_row_spec, _row_spec, _d_spec, _d_spec, _d_spec],
    out_shape=[_fmat, _fmat, _fcol, _fcol, _fcol],
)

_tc2_call = pl.pallas_call(
    _tc2_body,
    grid=(_NBLK,),
    in_specs=[_zp_spec, _row_spec, _d_spec, _w16_spec, _b16_spec],
    out_specs=[_row_spec, _d_spec, _d_spec],
    out_shape=[_fmat, _fcol, _fcol],
)

_tc3_call = pl.pallas_call(
    _tc3_body,
    grid=(_NBLK,),
    in_specs=[_zp_spec, _row_spec, _d_spec, _w_spec, _bias_spec],
    out_specs=_row_spec,
    out_shape=_fmat,
)


def kernel(h, edge_index, W1, b1, Wg, bg, W2, b2):
    f32 = jnp.float32
    npad_e = _EPAD - _E
    src_flat = jnp.pad(edge_index[0], (0, npad_e))
    dst_flat = jnp.pad(edge_index[1], (0, npad_e), constant_values=_N)
    h_pad = jnp.pad(h, ((0, _NPAD - _N), (0, 0)))
    b1r = b1.reshape(1, _H)

    wg16 = jnp.zeros((2, _H, 16), f32)
    wg16 = wg16.at[:, :, 0].set(Wg[:, :_H, 0]).at[:, :, 1].set(Wg[:, _H:, 0])
    bg16 = jnp.zeros((2, 1, 16), f32).at[:, 0, 0].set(bg[:, 0])
    W2p = jnp.zeros((_H, _H), f32).at[:, :_O].set(W2)
    b2r = jnp.zeros((1, _H), f32).at[0, :_O].set(b2)

    deg_kernel, layer_kernel = _sc_kernels()
    degp = deg_kernel(dst_flat.reshape(_NW, _NFC, _K))
    h1, hd1, a0, b0, d = _tc1_call(h_pad, W1, b1r, wg16[0], bg16[0], degp)
    zp0 = layer_kernel(src_flat, dst_flat,
                       a0.reshape(_NPAD), b0.reshape(_NPAD), hd1)
    hd2, a1, b1v = _tc2_call(zp0, h1, d, wg16[1], bg16[1])
    zp1 = layer_kernel(src_flat, dst_flat,
                       a1.reshape(_NPAD), b1v.reshape(_NPAD), hd2)
    outp = _tc3_call(zp1, h1, d, W2p, b2r)
    return outp[:_N, :_O]


# speedup vs baseline: 1.1328x; 1.1328x over previous
import functools

import jax
import jax.numpy as jnp
from jax import lax
from jax.experimental import pallas as pl
from jax.experimental.pallas import tpu as pltpu
from jax.experimental.pallas import tpu_sc as plsc

_N = 10000
_E = 320000
_H = 128
_O = 40
_EPS = 0.3
_NPAD = 10112
_NW = 32
_K = 64
_EPT = 10112
_EPAD = _NW * _EPT
_NFC = _EPT // _K
_RPW = _NPAD // 16
_BLK = 632
_NBLK = _NPAD // _BLK


@functools.cache
def _sc_kernels():
    mesh = plsc.VectorSubcoreMesh(core_axis_name="c", subcore_axis_name="s",
                                  num_cores=2, num_subcores=16)
    params = pltpu.CompilerParams(needs_layout_passes=False)
    deg = functools.partial(
        pl.kernel,
        out_type=jax.ShapeDtypeStruct((2, _NPAD, _H), jnp.float32),
        mesh=mesh,
        compiler_params=params,
        scratch_types=[
            pltpu.VMEM((_NFC, _K), jnp.int32),
            pltpu.VMEM((_K, _H), jnp.float32),
            pltpu.VMEM_SHARED((_NPAD, _H), jnp.float32),
        ],
    )(_deg_body)
    layer = functools.partial(
        pl.kernel,
        out_type=jax.ShapeDtypeStruct((2, _NPAD, _H), jnp.float32),
        mesh=mesh,
        compiler_params=params,
        scratch_types=[
            pltpu.VMEM((_NPAD,), jnp.float32),
            pltpu.VMEM((_NPAD,), jnp.float32),
            pltpu.VMEM((2, _K), jnp.int32),
            pltpu.VMEM((2, _K), jnp.int32),
            pltpu.VMEM((_K,), jnp.float32),
            pltpu.VMEM((_K, _H), jnp.float32),
            pltpu.VMEM((_K, _H), jnp.float32),
            pltpu.SemaphoreType.DMA,
            pltpu.SemaphoreType.DMA,
            pltpu.SemaphoreType.DMA,
            pltpu.SemaphoreType.DMA,
            pltpu.VMEM_SHARED((_NPAD, _H), jnp.float32),
        ],
    )(_layer_body)
    return deg, layer


def _deg_body(dst_hbm, out_hbm, idx_v, ones_v, deg_sh):
    c = lax.axis_index("c")
    s = lax.axis_index("s")
    wid = s * 2 + c
    pltpu.sync_copy(dst_hbm.at[wid], idx_v)

    def _fill(val):
        def _body(i, carry):
            for cb in range(_H // 16):
                ones_v[i, pl.ds(cb * 16, 16)] = jnp.full((16,), val,
                                                         jnp.float32)
            return carry
        lax.fori_loop(0, _K, _body, 0)

    row0 = s * _RPW
    _fill(0.0)

    def _zero(j, carry):
        pltpu.sync_copy(ones_v, deg_sh.at[pl.ds(row0 + j * _K, _K)])
        return carry

    lax.fori_loop(0, _RPW // _K, _zero, 0)
    pltpu.sync_copy(ones_v.at[pl.ds(0, _RPW % _K)],
                    deg_sh.at[pl.ds(row0 + (_RPW // _K) * _K, _RPW % _K)])
    _fill(1.0)
    plsc.subcore_barrier()

    def _scat(g, carry):
        pltpu.sync_copy(ones_v, deg_sh.at[idx_v.at[g]], add=True)
        return carry

    lax.fori_loop(0, _NFC, _scat, 0)
    plsc.subcore_barrier()
    pltpu.sync_copy(deg_sh.at[pl.ds(row0, _RPW)],
                    out_hbm.at[c, pl.ds(row0, _RPW)])


def _layer_body(ei_hbm, a_hbm, b_hbm, hd_hbm, out_hbm,
                a_v, b_v, idx0, idx1, e_v, rows0, rows1,
                gsem0, gsem1, ssem0, ssem1, z_sh):
    c = lax.axis_index("c")
    s = lax.axis_index("s")
    wid = s * 2 + c
    pltpu.sync_copy(a_hbm, a_v)
    pltpu.sync_copy(b_hbm, b_v)

    def _zrow(i, carry):
        for cb in range(_H // 16):
            rows0[i, pl.ds(cb * 16, 16)] = jnp.zeros((16,), jnp.float32)
        return carry

    lax.fori_loop(0, _K, _zrow, 0)
    row0 = s * _RPW

    def _zcp(j, carry):
        pltpu.sync_copy(rows0, z_sh.at[pl.ds(row0 + j * _K, _K)])
        return carry

    lax.fori_loop(0, _RPW // _K, _zcp, 0)
    pltpu.sync_copy(rows0.at[pl.ds(0, _RPW % _K)],
                    z_sh.at[pl.ds(row0 + (_RPW // _K) * _K, _RPW % _K)])

    def _fetch(cc, idx, rows, sem):
        pltpu.sync_copy(ei_hbm.at[wid, cc], idx)
        pltpu.async_copy(hd_hbm.at[idx.at[0]], rows, sem)

    _fetch(0, idx0, rows0, gsem0)
    _fetch(1, idx1, rows1, gsem1)
    plsc.subcore_barrier()

    def _gate(idx):
        for j in range(_K // 16):
            sv = idx[0, pl.ds(j * 16, 16)]
            dv = idx[1, pl.ds(j * 16, 16)]
            av = plsc.load_gather(a_v, [dv])
            bv = plsc.load_gather(b_v, [sv])
            x = av + bv
            t = jnp.exp(-2.0 * jnp.abs(x))
            e_v[pl.ds(j * 16, 16)] = jnp.sign(x) * (1.0 - t) / (1.0 + t)

    def _scale(rows):
        def _body(q, carry):
            for u in range(4):
                k = q * 4 + u
                ek = plsc.load_gather(e_v, [jnp.full((16,), k, jnp.int32)])
                for cb in range(_H // 16):
                    rows[k, pl.ds(cb * 16, 16)] = (
                        rows[k, pl.ds(cb * 16, 16)] * ek)
            return carry

        lax.fori_loop(0, _K // 4, _body, 0)

    def _pair(t, carry):
        c0 = 2 * t
        pltpu.make_async_copy(hd_hbm.at[idx0.at[0]], rows0, gsem0).wait()
        _gate(idx0)
        _scale(rows0)
        pltpu.sync_copy(rows0, z_sh.at[idx0.at[1]], add=True)

        @pl.when(c0 + 2 < _NFC)
        def _():
            _fetch(c0 + 2, idx0, rows0, gsem0)

        pltpu.make_async_copy(hd_hbm.at[idx1.at[0]], rows1, gsem1).wait()
        _gate(idx1)
        _scale(rows1)
        pltpu.sync_copy(rows1, z_sh.at[idx1.at[1]], add=True)

        @pl.when(c0 + 3 < _NFC)
        def _():
            _fetch(c0 + 3, idx1, rows1, gsem1)

        return carry

    lax.fori_loop(0, _NFC // 2, _pair, 0)

    plsc.subcore_barrier()
    pltpu.sync_copy(z_sh.at[pl.ds(row0, _RPW)],
                    out_hbm.at[c, pl.ds(row0, _RPW)])


def _tc1_body(h_ref, w1_ref, b1_ref, wg_ref, bg_ref, deg_ref,
              h1_ref, hd_ref, a_ref, b_ref, d_ref):
    x = h_ref[...]
    h1 = jnp.maximum(
        jnp.dot(x, w1_ref[...], preferred_element_type=jnp.float32)
        + b1_ref[...], 0.0)
    degb = deg_ref[...]
    deg = degb[0, :, 0:1] + degb[1, :, 0:1]
    d = jnp.where(deg > 0.0, lax.rsqrt(jnp.maximum(deg, 1.0)), 0.0)
    h1_ref[...] = h1
    hd_ref[...] = h1 * d
    d_ref[...] = d
    ab = jnp.dot(h1, wg_ref[...],
                 preferred_element_type=jnp.float32) + bg_ref[...]
    a_ref[...] = ab[:, 0:1]
    b_ref[...] = ab[:, 1:2]


def _tc2_body(zp_ref, h1_ref, d_ref, wg_ref, bg_ref, hd_ref, a_ref, b_ref):
    z = zp_ref[0] + zp_ref[1]
    d = d_ref[...]
    h2 = _EPS * h1_ref[...] + d * z
    hd_ref[...] = h2 * d
    ab = jnp.dot(h2, wg_ref[...],
                 preferred_element_type=jnp.float32) + bg_ref[...]
    a_ref[...] = ab[:, 0:1]
    b_ref[...] = ab[:, 1:2]


def _tc3_body(zp_ref, h1_ref, d_ref, w2_ref, b2_ref, out_ref):
    z = zp_ref[0] + zp_ref[1]
    h3 = _EPS * h1_ref[...] + d_ref[...] * z
    logits = jnp.dot(h3, w2_ref[...],
                     preferred_element_type=jnp.float32) + b2_ref[...]
    col = lax.broadcasted_iota(jnp.int32, (_BLK, _H), 1)
    logits = jnp.where(col < _O, logits, -1e30)
    m = jnp.max(logits, axis=1, keepdims=True)
    lse = jnp.log(jnp.sum(jnp.exp(logits - m), axis=1, keepdims=True))
    out_ref[...] = logits - m - lse


_row_spec = pl.BlockSpec((_BLK, _H), lambda i: (i, 0))
_w_spec = pl.BlockSpec((_H, _H), lambda i: (0, 0))
_bias_spec = pl.BlockSpec((1, _H), lambda i: (0, 0))
_w16_spec = pl.BlockSpec((_H, 16), lambda i: (0, 0))
_b16_spec = pl.BlockSpec((1, 16), lambda i: (0, 0))
_d_spec = pl.BlockSpec((_BLK, 1), lambda i: (i, 0))
_zp_spec = pl.BlockSpec((2, _BLK, _H), lambda i: (0, i, 0))
_fmat = jax.ShapeDtypeStruct((_NPAD, _H), jnp.float32)
_fcol = jax.ShapeDtypeStruct((_NPAD, 1), jnp.float32)

_tc1_call = pl.pallas_call(
    _tc1_body,
    grid=(_NBLK,),
    in_specs=[_row_spec, _w_spec, _bias_spec, _w16_spec, _b16_spec,
              _zp_spec],
    out_specs=[_row_spec, _row_spec, _d_spec, _d_spec, _d_spec],
    out_shape=[_fmat, _fmat, _fcol, _fcol, _fcol],
)

_tc2_call = pl.pallas_call(
    _tc2_body,
    grid=(_NBLK,),
    in_specs=[_zp_spec, _row_spec, _d_spec, _w16_spec, _b16_spec],
    out_specs=[_row_spec, _d_spec, _d_spec],
    out_shape=[_fmat, _fcol, _fcol],
)

_tc3_call = pl.pallas_call(
    _tc3_body,
    grid=(_NBLK,),
    in_specs=[_zp_spec, _row_spec, _d_spec, _w_spec, _bias_spec],
    out_specs=_row_spec,
    out_shape=_fmat,
)


def kernel(h, edge_index, W1, b1, Wg, bg, W2, b2):
    f32 = jnp.float32
    npad_e = _EPAD - _E
    src_flat = jnp.pad(edge_index[0], (0, npad_e))
    dst_flat = jnp.pad(edge_index[1], (0, npad_e), constant_values=_N)
    ei = jnp.stack([src_flat.reshape(_NW, _NFC, _K),
                    dst_flat.reshape(_NW, _NFC, _K)], axis=2)
    h_pad = jnp.pad(h, ((0, _NPAD - _N), (0, 0)))
    b1r = b1.reshape(1, _H)

    wg16 = jnp.zeros((2, _H, 16), f32)
    wg16 = wg16.at[:, :, 0].set(Wg[:, :_H, 0]).at[:, :, 1].set(Wg[:, _H:, 0])
    bg16 = jnp.zeros((2, 1, 16), f32).at[:, 0, 0].set(bg[:, 0])
    W2p = jnp.zeros((_H, _H), f32).at[:, :_O].set(W2)
    b2r = jnp.zeros((1, _H), f32).at[0, :_O].set(b2)

    deg_kernel, layer_kernel = _sc_kernels()
    degp = deg_kernel(dst_flat.reshape(_NW, _NFC, _K))
    h1, hd1, a0, b0, d = _tc1_call(h_pad, W1, b1r, wg16[0], bg16[0], degp)
    zp0 = layer_kernel(ei, a0.reshape(_NPAD), b0.reshape(_NPAD), hd1)
    hd2, a1, b1v = _tc2_call(zp0, h1, d, wg16[1], bg16[1])
    zp1 = layer_kernel(ei, a1.reshape(_NPAD), b1v.reshape(_NPAD), hd2)
    outp = _tc3_call(zp1, h1, d, W2p, b2r)
    return outp[:_N, :_O]


# speedup vs baseline: 1.2513x; 1.1046x over previous
import functools

import jax
import jax.numpy as jnp
from jax import lax
from jax.experimental import pallas as pl
from jax.experimental.pallas import tpu as pltpu
from jax.experimental.pallas import tpu_sc as plsc

_N = 10000
_E = 320000
_H = 128
_O = 40
_EPS = 0.3
_NPAD = 10112
_NW = 32
_EPT = _E // _NW
_K = 64
_NFC = _EPT // _K
_TK = _EPT - _NFC * _K
_DK = 80
_DCH = _EPT // _DK
_RPW = _NPAD // 16
_BLK = 632
_NBLK = _NPAD // _BLK


@functools.cache
def _sc_kernels():
    mesh = plsc.VectorSubcoreMesh(core_axis_name="c", subcore_axis_name="s",
                                  num_cores=2, num_subcores=16)
    params = pltpu.CompilerParams(needs_layout_passes=False)
    deg = functools.partial(
        pl.kernel,
        out_type=jax.ShapeDtypeStruct((2, _NPAD, _H), jnp.float32),
        mesh=mesh,
        compiler_params=params,
        scratch_types=[
            pltpu.VMEM((_DCH, _DK), jnp.int32),
            pltpu.VMEM((_DK, _H), jnp.float32),
            pltpu.VMEM((_DK, _H), jnp.float32),
            pltpu.VMEM_SHARED((_NPAD, _H), jnp.float32),
        ],
    )(_deg_body)
    layer = functools.partial(
        pl.kernel,
        out_type=jax.ShapeDtypeStruct((2, _NPAD, _H), jnp.float32),
        mesh=mesh,
        compiler_params=params,
        scratch_types=[
            pltpu.VMEM((_N,), jnp.float32),
            pltpu.VMEM((_N,), jnp.float32),
            pltpu.VMEM((2, _K), jnp.int32),
            pltpu.VMEM((2, _K), jnp.int32),
            pltpu.VMEM((2, _TK), jnp.int32),
            pltpu.VMEM((_K,), jnp.float32),
            pltpu.VMEM((_K, _H), jnp.float32),
            pltpu.VMEM((_K, _H), jnp.float32),
            pltpu.SemaphoreType.DMA,
            pltpu.SemaphoreType.DMA,
            pltpu.VMEM_SHARED((_NPAD, _H), jnp.float32),
        ],
    )(_layer_body)
    return deg, layer


def _deg_body(dst_hbm, out_hbm, dst_v, ones_v, zeros_v, deg_sh):
    c = lax.axis_index("c")
    s = lax.axis_index("s")
    wid = s * 2 + c
    pltpu.sync_copy(dst_hbm.at[wid], dst_v)

    def _fill(i, carry):
        for cb in range(_H // 16):
            ones_v[i, pl.ds(cb * 16, 16)] = jnp.ones((16,), jnp.float32)
            zeros_v[i, pl.ds(cb * 16, 16)] = jnp.zeros((16,), jnp.float32)
        return carry

    lax.fori_loop(0, _DK, _fill, 0)

    row0 = s * _RPW

    def _zero(j, carry):
        pltpu.sync_copy(zeros_v, deg_sh.at[pl.ds(row0 + j * _DK, _DK)])
        return carry

    lax.fori_loop(0, _RPW // _DK, _zero, 0)
    pltpu.sync_copy(zeros_v.at[pl.ds(0, _RPW % _DK)],
                    deg_sh.at[pl.ds(row0 + (_RPW // _DK) * _DK, _RPW % _DK)])
    plsc.subcore_barrier()

    def _scat(g, carry):
        pltpu.sync_copy(ones_v, deg_sh.at[dst_v.at[g]], add=True)
        return carry

    lax.fori_loop(0, _DCH, _scat, 0)
    plsc.subcore_barrier()
    pltpu.sync_copy(deg_sh.at[pl.ds(row0, _RPW)],
                    out_hbm.at[c, pl.ds(row0, _RPW)])


def _layer_body(src_hbm, dst_hbm, a_hbm, b_hbm, hd_hbm, out_hbm,
                a_v, b_v, idx0, idx1, idxt, e_v, rows0, rows1,
                sem0, sem1, z_sh):
    c = lax.axis_index("c")
    s = lax.axis_index("s")
    wid = s * 2 + c
    ebase = wid * _EPT
    pltpu.sync_copy(a_hbm.at[pl.ds(0, _N)], a_v)
    pltpu.sync_copy(b_hbm.at[pl.ds(0, _N)], b_v)

    def _zrow(i, carry):
        for cb in range(_H // 16):
            rows0[i, pl.ds(cb * 16, 16)] = jnp.zeros((16,), jnp.float32)
        return carry

    lax.fori_loop(0, _K, _zrow, 0)
    row0 = s * _RPW

    def _zcp(j, carry):
        pltpu.sync_copy(rows0, z_sh.at[pl.ds(row0 + j * _K, _K)])
        return carry

    lax.fori_loop(0, _RPW // _K, _zcp, 0)
    pltpu.sync_copy(rows0.at[pl.ds(0, _RPW % _K)],
                    z_sh.at[pl.ds(row0 + (_RPW // _K) * _K, _RPW % _K)])

    def _fetch(cc, idx, rows, sem):
        pltpu.sync_copy(src_hbm.at[pl.ds(ebase + cc * _K, _K)], idx.at[0])
        pltpu.sync_copy(dst_hbm.at[pl.ds(ebase + cc * _K, _K)], idx.at[1])
        pltpu.async_copy(hd_hbm.at[idx.at[0]], rows, sem)

    _fetch(0, idx0, rows0, sem0)
    _fetch(1, idx1, rows1, sem1)
    plsc.subcore_barrier()

    def _gate(idx, rows, ngroups):
        for j in range(ngroups):
            sv = idx[0, pl.ds(j * 16, 16)]
            dv = idx[1, pl.ds(j * 16, 16)]
            av = plsc.load_gather(a_v, [dv])
            bv = plsc.load_gather(b_v, [sv])
            x = av + bv
            t = jnp.exp(-2.0 * jnp.abs(x))
            e_v[pl.ds(j * 16, 16)] = jnp.sign(x) * (1.0 - t) / (1.0 + t)

        def _scale(q, carry):
            for u in range(4):
                k = q * 4 + u
                ek = plsc.load_gather(e_v, [jnp.full((16,), k, jnp.int32)])
                for cb in range(_H // 16):
                    rows[k, pl.ds(cb * 16, 16)] = (
                        rows[k, pl.ds(cb * 16, 16)] * ek)
            return carry

        lax.fori_loop(0, ngroups * 4, _scale, 0)

    def _pair(t, carry):
        c0 = 2 * t
        pltpu.make_async_copy(hd_hbm.at[idx0.at[0]], rows0, sem0).wait()
        _gate(idx0, rows0, _K // 16)
        pltpu.sync_copy(rows0, z_sh.at[idx0.at[1]], add=True)

        @pl.when(c0 + 2 < _NFC)
        def _():
            _fetch(c0 + 2, idx0, rows0, sem0)

        pltpu.make_async_copy(hd_hbm.at[idx1.at[0]], rows1, sem1).wait()
        _gate(idx1, rows1, _K // 16)
        pltpu.sync_copy(rows1, z_sh.at[idx1.at[1]], add=True)

        @pl.when(c0 + 3 < _NFC)
        def _():
            _fetch(c0 + 3, idx1, rows1, sem1)

        return carry

    lax.fori_loop(0, _NFC // 2, _pair, 0)

    pltpu.sync_copy(src_hbm.at[pl.ds(ebase + _NFC * _K, _TK)], idxt.at[0])
    pltpu.sync_copy(dst_hbm.at[pl.ds(ebase + _NFC * _K, _TK)], idxt.at[1])
    pltpu.sync_copy(hd_hbm.at[idxt.at[0]], rows0.at[pl.ds(0, _TK)])
    _gate(idxt, rows0, _TK // 16)
    pltpu.sync_copy(rows0.at[pl.ds(0, _TK)], z_sh.at[idxt.at[1]], add=True)

    plsc.subcore_barrier()
    pltpu.sync_copy(z_sh.at[pl.ds(row0, _RPW)],
                    out_hbm.at[c, pl.ds(row0, _RPW)])


def _tc1_body(h_ref, w1_ref, b1_ref, wg_ref, bg_ref, deg_ref,
              h1_ref, hd_ref, a_ref, b_ref, d_ref):
    x = h_ref[...]
    h1 = jnp.maximum(
        jnp.dot(x, w1_ref[...], preferred_element_type=jnp.float32)
        + b1_ref[...], 0.0)
    degb = deg_ref[...]
    deg = degb[0, :, 0:1] + degb[1, :, 0:1]
    d = jnp.where(deg > 0.0, lax.rsqrt(jnp.maximum(deg, 1.0)), 0.0)
    h1_ref[...] = h1
    hd_ref[...] = h1 * d
    d_ref[...] = d
    ab = jnp.dot(h1, wg_ref[...],
                 preferred_element_type=jnp.float32) + bg_ref[...]
    a_ref[...] = ab[:, 0:1]
    b_ref[...] = ab[:, 1:2]


def _tc2_body(zp_ref, h1_ref, d_ref, wg_ref, bg_ref, hd_ref, a_ref, b_ref):
    z = zp_ref[0] + zp_ref[1]
    d = d_ref[...]
    h2 = _EPS * h1_ref[...] + d * z
    hd_ref[...] = h2 * d
    ab = jnp.dot(h2, wg_ref[...],
                 preferred_element_type=jnp.float32) + bg_ref[...]
    a_ref[...] = ab[:, 0:1]
    b_ref[...] = ab[:, 1:2]


def _tc3_body(zp_ref, h1_ref, d_ref, w2_ref, b2_ref, out_ref):
    z = zp_ref[0] + zp_ref[1]
    h3 = _EPS * h1_ref[...] + d_ref[...] * z
    logits = jnp.dot(h3, w2_ref[...],
                     preferred_element_type=jnp.float32) + b2_ref[...]
    col = lax.broadcasted_iota(jnp.int32, (_BLK, _H), 1)
    logits = jnp.where(col < _O, logits, -1e30)
    m = jnp.max(logits, axis=1, keepdims=True)
    lse = jnp.log(jnp.sum(jnp.exp(logits - m), axis=1, keepdims=True))
    out_ref[...] = logits - m - lse


_row_spec = pl.BlockSpec((_BLK, _H), lambda i: (i, 0))
_w_spec = pl.BlockSpec((_H, _H), lambda i: (0, 0))
_bias_spec = pl.BlockSpec((1, _H), lambda i: (0, 0))
_w16_spec = pl.BlockSpec((_H, 16), lambda i: (0, 0))
_b16_spec = pl.BlockSpec((1, 16), lambda i: (0, 0))
_d_spec = pl.BlockSpec((_BLK, 1), lambda i: (i, 0))
_zp_spec = pl.BlockSpec((2, _BLK, _H), lambda i: (0, i, 0))
_fmat = jax.ShapeDtypeStruct((_NPAD, _H), jnp.float32)
_fcol = jax.ShapeDtypeStruct((_NPAD, 1), jnp.float32)

_tc1_call = pl.pallas_call(
    _tc1_body,
    grid=(_NBLK,),
    in_specs=[_row_spec, _w_spec, _bias_spec, _w16_spec, _b16_spec,
              _zp_spec],
    out_specs=[_row_spec, _row_spec, _d_spec, _d_spec, _d_spec],
    out_shape=[_fmat, _fmat, _fcol, _fcol, _fcol],
)

_tc2_call = pl.pallas_call(
    _tc2_body,
    grid=(_NBLK,),
    in_specs=[_zp_spec, _row_spec, _d_spec, _w16_spec, _b16_spec],
    out_specs=[_row_spec, _d_spec, _d_spec],
    out_shape=[_fmat, _fcol, _fcol],
)

_tc3_call = pl.pallas_call(
    _tc3_body,
    grid=(_NBLK,),
    in_specs=[_zp_spec, _row_spec, _d_spec, _w_spec, _bias_spec],
    out_specs=_row_spec,
    out_shape=_fmat,
)


def kernel(h, edge_index, W1, b1, Wg, bg, W2, b2):
    f32 = jnp.float32
    src_flat = edge_index[0]
    dst_flat = edge_index[1]
    h_pad = jnp.pad(h, ((0, _NPAD - _N), (0, 0)))
    b1r = b1.reshape(1, _H)

    wg16 = jnp.zeros((2, _H, 16), f32)
    wg16 = wg16.at[:, :, 0].set(Wg[:, :_H, 0]).at[:, :, 1].set(Wg[:, _H:, 0])
    bg16 = jnp.zeros((2, 1, 16), f32).at[:, 0, 0].set(bg[:, 0])
    W2p = jnp.zeros((_H, _H), f32).at[:, :_O].set(W2)
    b2r = jnp.zeros((1, _H), f32).at[0, :_O].set(b2)

    deg_kernel, layer_kernel = _sc_kernels()
    degp = deg_kernel(dst_flat.reshape(_NW, _DCH, _DK))
    h1, hd1, a0, b0, d = _tc1_call(h_pad, W1, b1r, wg16[0], bg16[0], degp)
    zp0 = layer_kernel(src_flat, dst_flat,
                       a0.reshape(_NPAD), b0.reshape(_NPAD), hd1)
    hd2, a1, b1v = _tc2_call(zp0, h1, d, wg16[1], bg16[1])
    zp1 = layer_kernel(src_flat, dst_flat,
                       a1.reshape(_NPAD), b1v.reshape(_NPAD), hd2)
    outp = _tc3_call(zp1, h1, d, W2p, b2r)
    return outp[:_N, :_O]
